# Initial kernel scaffold; baseline (speedup 1.0000x reference)
import functools

import jax
import jax.numpy as jnp
from jax import lax
from jax.experimental import pallas as pl
from jax.experimental.pallas import tpu as pltpu
from jax.experimental.pallas import tpu_sc as plsc

NA = 100000
NE = 6400000
ROW = 128
NROWS = NE // ROW
NBC = 16
NCHUNKS = NROWS // NBC
NC = 2
NS = 16
NW = NC * NS
NT = -(-NCHUNKS // NW)
ZR = 6248
ZR_LAST = NA - (NS - 1) * ZR


def _sc_body(dx_h, pi_h, pj_h, z_h, outp_h, outn_h,
             dxb, pib, pjb, accp, accn, sem):
    c = lax.axis_index("c")
    s = lax.axis_index("s")
    w = s * NC + c

    @pl.when(s < NS - 1)
    def _():
        pltpu.sync_copy(z_h.at[pl.ds(s * ZR, ZR)], accp.at[pl.ds(s * ZR, ZR)])
        pltpu.sync_copy(z_h.at[pl.ds(s * ZR, ZR)], accn.at[pl.ds(s * ZR, ZR)])

    @pl.when(s == NS - 1)
    def _():
        base = (NS - 1) * ZR
        pltpu.sync_copy(z_h.at[pl.ds(base, ZR_LAST)],
                        accp.at[pl.ds(base, ZR_LAST)])
        pltpu.sync_copy(z_h.at[pl.ds(base, ZR_LAST)],
                        accn.at[pl.ds(base, ZR_LAST)])

    plsc.subcore_barrier()

    def chunk_body(t, carry):
        k = t * NW + w

        @pl.when(k < NCHUNKS)
        def _():
            pltpu.sync_copy(dx_h.at[pl.ds(k * NBC, NBC)], dxb)
            pltpu.sync_copy(pi_h.at[pl.ds(k * NBC, NBC)], pib)
            pltpu.sync_copy(pj_h.at[pl.ds(k * NBC, NBC)], pjb)
            descs = []
            for b in range(NBC):
                descs.append(pltpu.async_copy(
                    dxb.at[b], accp.at[pib.at[b]], sem, add=True))
                descs.append(pltpu.async_copy(
                    dxb.at[b], accn.at[pjb.at[b]], sem, add=True))
            for d in descs:
                d.wait()

        return carry

    lax.fori_loop(0, NT, chunk_body, 0)

    plsc.subcore_barrier()

    @pl.when(s < NS - 1)
    def _():
        pltpu.sync_copy(accp.at[pl.ds(s * ZR, ZR)],
                        outp_h.at[c, pl.ds(s * ZR, ZR)])
        pltpu.sync_copy(accn.at[pl.ds(s * ZR, ZR)],
                        outn_h.at[c, pl.ds(s * ZR, ZR)])

    @pl.when(s == NS - 1)
    def _():
        base = (NS - 1) * ZR
        pltpu.sync_copy(accp.at[pl.ds(base, ZR_LAST)],
                        outp_h.at[c, pl.ds(base, ZR_LAST)])
        pltpu.sync_copy(accn.at[pl.ds(base, ZR_LAST)],
                        outn_h.at[c, pl.ds(base, ZR_LAST)])


_sc_scatter = functools.partial(
    pl.kernel,
    out_type=[
        jax.ShapeDtypeStruct((NC, NA, 3), jnp.float32),
        jax.ShapeDtypeStruct((NC, NA, 3), jnp.float32),
    ],
    mesh=plsc.VectorSubcoreMesh(core_axis_name="c", subcore_axis_name="s"),
    scratch_types=[
        pltpu.VMEM((NBC, ROW, 3), jnp.float32),
        pltpu.VMEM((NBC, ROW), jnp.int32),
        pltpu.VMEM((NBC, ROW), jnp.int32),
        pltpu.VMEM_SHARED((NA, 3), jnp.float32),
        pltpu.VMEM_SHARED((NA, 3), jnp.float32),
        pltpu.SemaphoreType.DMA,
    ],
)(_sc_body)


def _combine_body(p_ref, n_ref, o_ref):
    o_ref[...] = (p_ref[0] - n_ref[0]) + (p_ref[1] - n_ref[1])


_combine = pl.pallas_call(
    _combine_body,
    out_shape=jax.ShapeDtypeStruct((NA * 3,), jnp.float32),
)


def kernel(dx, pair_i, pair_j):
    dx3 = dx.reshape(NROWS, ROW, 3)
    pi2 = pair_i.reshape(NROWS, ROW)
    pj2 = pair_j.reshape(NROWS, ROW)
    zeros = jnp.zeros((NA, 3), jnp.float32)
    part_p, part_n = _sc_scatter(dx3, pi2, pj2, zeros)
    flat = _combine(part_p.reshape(NC, NA * 3), part_n.reshape(NC, NA * 3))
    return flat.reshape(NA, 3)


# speedup vs baseline: 15.7927x; 15.7927x over previous
import functools

import jax
import jax.numpy as jnp
from jax import lax
from jax.experimental import pallas as pl
from jax.experimental.pallas import tpu as pltpu
from jax.experimental.pallas import tpu_sc as plsc

NA = 100000
NE = 6400000
C = 2048
NCHUNKS = NE // C
NC = 2
NS = 16
NW = NC * NS
NT = -(-NCHUNKS // NW)
ZR = 6256
ZR_LAST = NA - (NS - 1) * ZR


def _sc_body(pi_h, pj_h, dxt_h, z_h, outp_h, outn_h,
             pib, pjb, dxb,
             px, py, pz, nx, ny, nz, sem):
    c = lax.axis_index("c")
    s = lax.axis_index("s")
    w = s * NC + c
    planes_p = (px, py, pz)
    planes_n = (nx, ny, nz)

    def init(off, ln):
        for a in planes_p + planes_n:
            pltpu.sync_copy(z_h.at[pl.ds(off, ln)], a.at[pl.ds(off, ln)])

    @pl.when(s < NS - 1)
    def _():
        init(s * ZR, ZR)

    @pl.when(s == NS - 1)
    def _():
        init((NS - 1) * ZR, ZR_LAST)

    plsc.subcore_barrier()

    def chunk_body(t, carry):
        k = t * NW + w

        @pl.when(k < NCHUNKS)
        def _():
            e0 = k * C
            pltpu.sync_copy(pi_h.at[pl.ds(e0, C)], pib)
            pltpu.sync_copy(pj_h.at[pl.ds(e0, C)], pjb)
            for j in range(3):
                pltpu.sync_copy(dxt_h.at[j, pl.ds(e0, C)], dxb.at[j])
            descs = []
            for j in range(3):
                descs.append(pltpu.async_copy(
                    dxb.at[j], planes_p[j].at[pib], sem, add=True))
                descs.append(pltpu.async_copy(
                    dxb.at[j], planes_n[j].at[pjb], sem, add=True))
            for d in descs:
                d.wait()

        return carry

    lax.fori_loop(0, NT, chunk_body, 0)

    plsc.subcore_barrier()

    def drain(off, ln):
        for j in range(3):
            pltpu.sync_copy(planes_p[j].at[pl.ds(off, ln)],
                            outp_h.at[c, j, pl.ds(off, ln)])
            pltpu.sync_copy(planes_n[j].at[pl.ds(off, ln)],
                            outn_h.at[c, j, pl.ds(off, ln)])

    @pl.when(s < NS - 1)
    def _():
        drain(s * ZR, ZR)

    @pl.when(s == NS - 1)
    def _():
        drain((NS - 1) * ZR, ZR_LAST)


_sc_scatter = functools.partial(
    pl.kernel,
    out_type=[
        jax.ShapeDtypeStruct((NC, 3, NA), jnp.float32),
        jax.ShapeDtypeStruct((NC, 3, NA), jnp.float32),
    ],
    mesh=plsc.VectorSubcoreMesh(core_axis_name="c", subcore_axis_name="s"),
    compiler_params=pltpu.CompilerParams(use_tc_tiling_on_sc=False),
    scratch_types=[
        pltpu.VMEM((C,), jnp.int32),
        pltpu.VMEM((C,), jnp.int32),
        pltpu.VMEM((3, C), jnp.float32),
        pltpu.VMEM_SHARED((NA,), jnp.float32),
        pltpu.VMEM_SHARED((NA,), jnp.float32),
        pltpu.VMEM_SHARED((NA,), jnp.float32),
        pltpu.VMEM_SHARED((NA,), jnp.float32),
        pltpu.VMEM_SHARED((NA,), jnp.float32),
        pltpu.VMEM_SHARED((NA,), jnp.float32),
        pltpu.SemaphoreType.DMA,
    ],
)(_sc_body)


_CB = 8192


def _combine_body(p_ref, n_ref, o_ref):
    o_ref[...] = (p_ref[0:3] + p_ref[3:6]) - (n_ref[0:3] + n_ref[3:6])


_combine = pl.pallas_call(
    _combine_body,
    grid=(-(-NA // _CB),),
    in_specs=[
        pl.BlockSpec((NC * 3, _CB), lambda i: (0, i)),
        pl.BlockSpec((NC * 3, _CB), lambda i: (0, i)),
    ],
    out_specs=pl.BlockSpec((3, _CB), lambda i: (0, i)),
    out_shape=jax.ShapeDtypeStruct((3, NA), jnp.float32),
)


def kernel(dx, pair_i, pair_j):
    dxt = dx.T
    zeros = jnp.zeros((NA,), jnp.float32)
    part_p, part_n = _sc_scatter(pair_i, pair_j, dxt, zeros)
    planes = _combine(part_p.reshape(NC * 3, NA), part_n.reshape(NC * 3, NA))
    return planes.T
